# baseline (speedup 1.0000x reference)
import functools

import jax
import jax.numpy as jnp
from jax import lax
from jax.experimental import pallas as pl
from jax.experimental.pallas import tpu as pltpu
from jax.experimental.pallas import tpu_sc as plsc

VOCAB = 100000
EMBED = 64
CONTEXT = 5
BATCH = 1024
HIDDEN = 128

NCTX = 2 * CONTEXT
TILE_V = 5120
N_TILES = (VOCAB + TILE_V - 1) // TILE_V

_info = plsc.get_sparse_core_info()
_NC, _NS = _info.num_cores, _info.num_subcores
_NW = _NC * _NS
_DPW = EMBED // _NW


def _sc_gather_t(idxT_hbm, tableT_hbm, embT_hbm, idx_v, row_v, out_v,
                 sem_row, sem_out):
    wid = lax.axis_index("s") * _NC + lax.axis_index("c")
    d0 = wid * _DPW
    row_cp = pltpu.async_copy(tableT_hbm.at[d0], row_v, sem_row)
    pltpu.sync_copy(idxT_hbm, idx_v)
    row_cp.wait()
    pend = [None, None]
    for r in range(_DPW):
        d = d0 + r
        for c in range(NCTX):
            b = c % 2
            if pend[b] is not None:
                pend[b].wait()
                pend[b] = None
            def gather_chunk(j, _):
                sl = pl.ds(j * 16, 16)
                out_v[b, sl] = plsc.load_gather(row_v, [idx_v[c, sl]])
                return 0

            lax.fori_loop(0, BATCH // 16, gather_chunk, 0, unroll=16)
            if r + 1 < _DPW and c == NCTX - 1:
                row_cp = pltpu.async_copy(tableT_hbm.at[d + 1], row_v, sem_row)
            pend[b] = pltpu.async_copy(
                out_v.at[b], embT_hbm.at[c * EMBED + d], sem_out)
        if r + 1 < _DPW:
            row_cp.wait()
    for p in pend:
        if p is not None:
            p.wait()


_gather_call = functools.partial(
    pl.kernel,
    mesh=plsc.VectorSubcoreMesh(core_axis_name="c", subcore_axis_name="s"),
    compiler_params=pltpu.CompilerParams(needs_layout_passes=False),
    out_type=jax.ShapeDtypeStruct((NCTX * EMBED, BATCH), jnp.float32),
    scratch_types=[
        pltpu.VMEM((NCTX, BATCH), jnp.int32),
        pltpu.VMEM((VOCAB,), jnp.float32),
        pltpu.VMEM((2, BATCH), jnp.float32),
        pltpu.SemaphoreType.DMA,
        pltpu.SemaphoreType.DMA,
    ],
)(_sc_gather_t)


def _mlp_body(embT_ref, w1t_ref, b1_ref, w2t_ref, b2t_ref, outT_ref, h_ref):
    @pl.when(pl.program_id(0) == 0)
    def _():
        hT = jnp.dot(w1t_ref[...].astype(jnp.bfloat16),
                     embT_ref[...].astype(jnp.bfloat16),
                     preferred_element_type=jnp.float32)
        h_ref[...] = jnp.maximum(hT + b1_ref[...], 0.0).astype(jnp.bfloat16)

    j = pl.program_id(0)
    lanes = lax.broadcasted_iota(jnp.int32, (TILE_V, N_TILES), 1)
    b2col = jnp.sum(jnp.where(lanes == j, b2t_ref[...], 0.0),
                    axis=1, keepdims=True)
    outT_ref[...] = (
        jnp.dot(w2t_ref[...].astype(jnp.bfloat16), h_ref[...],
                preferred_element_type=jnp.float32)
        + b2col
    )


def _mlp(embT, W1T, b1c, W2T, b2t):
    return pl.pallas_call(
        _mlp_body,
        grid=(N_TILES,),
        in_specs=[
            pl.BlockSpec((NCTX * EMBED, BATCH), lambda j: (0, 0)),
            pl.BlockSpec((HIDDEN, NCTX * EMBED), lambda j: (0, 0)),
            pl.BlockSpec((HIDDEN, 1), lambda j: (0, 0)),
            pl.BlockSpec((TILE_V, HIDDEN), lambda j: (j, 0)),
            pl.BlockSpec((TILE_V, N_TILES), lambda j: (0, 0)),
        ],
        out_specs=pl.BlockSpec((TILE_V, BATCH), lambda j: (j, 0)),
        out_shape=jax.ShapeDtypeStruct((VOCAB, BATCH), jnp.float32),
        scratch_shapes=[pltpu.VMEM((HIDDEN, BATCH), jnp.bfloat16)],
        compiler_params=pltpu.CompilerParams(
            dimension_semantics=("arbitrary",),
        ),
    )(embT, W1T, b1c, W2T, b2t)


def kernel(inputs, table, W1, b1, W2, b2):
    idxT = inputs.T.astype(jnp.int32)
    tableT = table.T
    embT = _gather_call(idxT, tableT)
    W2T = W2.T
    b2t = jnp.pad(b2, (0, N_TILES * TILE_V - VOCAB)).reshape(N_TILES, TILE_V).T
    outT = _mlp(embT, W1.T, b1.reshape(HIDDEN, 1), W2T, b2t)
    return outT.T


# speedup vs baseline: 1.0027x; 1.0027x over previous
import functools

import jax
import jax.numpy as jnp
from jax import lax
from jax.experimental import pallas as pl
from jax.experimental.pallas import tpu as pltpu
from jax.experimental.pallas import tpu_sc as plsc

VOCAB = 100000
EMBED = 64
CONTEXT = 5
BATCH = 1024
HIDDEN = 128

NCTX = 2 * CONTEXT
TILE_V = 5376
N_TILES = (VOCAB + TILE_V - 1) // TILE_V

_info = plsc.get_sparse_core_info()
_NC, _NS = _info.num_cores, _info.num_subcores
_NW = _NC * _NS
_DPW = EMBED // _NW


def _sc_gather_t(idxT_hbm, tableT_hbm, embT_hbm, idx_v, row_v, out_v,
                 sem_row, sem_out):
    wid = lax.axis_index("s") * _NC + lax.axis_index("c")
    d0 = wid * _DPW
    row_cp = pltpu.async_copy(tableT_hbm.at[d0], row_v, sem_row)
    pltpu.sync_copy(idxT_hbm, idx_v)
    row_cp.wait()
    pend = [None, None]
    for r in range(_DPW):
        d = d0 + r
        for c in range(NCTX):
            b = c % 2
            if pend[b] is not None:
                pend[b].wait()
                pend[b] = None
            def gather_chunk(j, _):
                sl = pl.ds(j * 16, 16)
                out_v[b, sl] = plsc.load_gather(row_v, [idx_v[c, sl]])
                return 0

            lax.fori_loop(0, BATCH // 16, gather_chunk, 0, unroll=8)
            if r + 1 < _DPW and c == NCTX - 1:
                row_cp = pltpu.async_copy(tableT_hbm.at[d + 1], row_v, sem_row)
            pend[b] = pltpu.async_copy(
                out_v.at[b], embT_hbm.at[c * EMBED + d], sem_out)
        if r + 1 < _DPW:
            row_cp.wait()
    for p in pend:
        if p is not None:
            p.wait()


_gather_call = functools.partial(
    pl.kernel,
    mesh=plsc.VectorSubcoreMesh(core_axis_name="c", subcore_axis_name="s"),
    compiler_params=pltpu.CompilerParams(needs_layout_passes=False),
    out_type=jax.ShapeDtypeStruct((NCTX * EMBED, BATCH), jnp.float32),
    scratch_types=[
        pltpu.VMEM((NCTX, BATCH), jnp.int32),
        pltpu.VMEM((VOCAB,), jnp.float32),
        pltpu.VMEM((2, BATCH), jnp.float32),
        pltpu.SemaphoreType.DMA,
        pltpu.SemaphoreType.DMA,
    ],
)(_sc_gather_t)


def _mlp_body(embT_ref, w1t_ref, b1_ref, w2t_ref, b2t_ref, outT_ref, h_ref):
    @pl.when(pl.program_id(0) == 0)
    def _():
        hT = jnp.dot(w1t_ref[...].astype(jnp.bfloat16),
                     embT_ref[...].astype(jnp.bfloat16),
                     preferred_element_type=jnp.float32)
        h_ref[...] = jnp.maximum(hT + b1_ref[...], 0.0).astype(jnp.bfloat16)

    j = pl.program_id(0)
    lanes = lax.broadcasted_iota(jnp.int32, (TILE_V, N_TILES), 1)
    b2col = jnp.sum(jnp.where(lanes == j, b2t_ref[...], 0.0),
                    axis=1, keepdims=True)
    outT_ref[...] = (
        jnp.dot(w2t_ref[...].astype(jnp.bfloat16), h_ref[...],
                preferred_element_type=jnp.float32)
        + b2col
    )


def _mlp(embT, W1T, b1c, W2T, b2t):
    return pl.pallas_call(
        _mlp_body,
        grid=(N_TILES,),
        in_specs=[
            pl.BlockSpec((NCTX * EMBED, BATCH), lambda j: (0, 0)),
            pl.BlockSpec((HIDDEN, NCTX * EMBED), lambda j: (0, 0)),
            pl.BlockSpec((HIDDEN, 1), lambda j: (0, 0)),
            pl.BlockSpec((TILE_V, HIDDEN), lambda j: (j, 0)),
            pl.BlockSpec((TILE_V, N_TILES), lambda j: (0, 0)),
        ],
        out_specs=pl.BlockSpec((TILE_V, BATCH), lambda j: (j, 0)),
        out_shape=jax.ShapeDtypeStruct((VOCAB, BATCH), jnp.float32),
        scratch_shapes=[pltpu.VMEM((HIDDEN, BATCH), jnp.bfloat16)],
        compiler_params=pltpu.CompilerParams(
            dimension_semantics=("arbitrary",),
        ),
    )(embT, W1T, b1c, W2T, b2t)


def kernel(inputs, table, W1, b1, W2, b2):
    idxT = inputs.T.astype(jnp.int32)
    tableT = table.T
    embT = _gather_call(idxT, tableT)
    W2T = W2.T
    b2t = jnp.pad(b2, (0, N_TILES * TILE_V - VOCAB)).reshape(N_TILES, TILE_V).T
    outT = _mlp(embT, W1.T, b1.reshape(HIDDEN, 1), W2T, b2t)
    return outT.T
